# Initial kernel scaffold; baseline (speedup 1.0000x reference)
import jax
import jax.numpy as jnp
from jax.experimental import pallas as pl


def kernel(q, k, v, g_cmp, g_slc, x_offsets):
    raise NotImplementedError("write your pallas kernel here")


# speedup vs baseline: 630.9445x; 630.9445x over previous
import functools

import jax
import jax.numpy as jnp
from jax.experimental import pallas as pl
from jax.experimental.pallas import tpu as pltpu

_BLOCK_SIZE = 32
_BLOCK_COUNTS = 4


def _hstu_bsa_kernel(q_ref, k_ref, v_ref, gc_ref, gs_ref, out_ref, *, bs, ssel):
    f32 = jnp.float32
    q = q_ref[0, 0]
    k = k_ref[0, 0]
    v = v_ref[0, 0]
    gc = gc_ref[0, 0]
    gs = gs_ref[0, 0]
    L, D = q.shape
    nblk = L // bs
    scale = D ** (-0.5)

    tok = jax.lax.broadcasted_iota(jnp.int32, (nblk, L), 1)
    blk = jax.lax.broadcasted_iota(jnp.int32, (nblk, L), 0)
    pool = (tok // bs == blk).astype(f32)

    hi = jax.lax.Precision.HIGHEST
    k_cmp = jnp.dot(pool, k, preferred_element_type=f32,
                    precision=hi) * (1.0 / bs)
    v_cmp = jnp.dot(pool, v, preferred_element_type=f32,
                    precision=hi) * (1.0 / bs)

    bf = jnp.bfloat16
    s_cmp = jnp.dot(q.astype(bf), k_cmp.T.astype(bf),
                    preferred_element_type=f32) * scale
    qrow = jax.lax.broadcasted_iota(jnp.int32, (L, nblk), 0)
    kcol = jax.lax.broadcasted_iota(jnp.int32, (L, nblk), 1)
    causal_blk = (qrow // bs) >= kcol
    s_sel = jnp.where(causal_blk, s_cmp, -jnp.inf)

    rank = jnp.zeros((L, nblk), dtype=f32)
    for j in range(nblk):
        sj = s_sel[:, j:j + 1]
        beats = (sj > s_sel) | ((sj == s_sel) & (j < kcol))
        rank = rank + beats.astype(f32)
    selected = (rank < ssel).astype(f32)

    mask_tok = jnp.dot(selected, pool, preferred_element_type=f32)
    qt = jax.lax.broadcasted_iota(jnp.int32, (L, L), 0)
    kt = jax.lax.broadcasted_iota(jnp.int32, (L, L), 1)
    mask_tok = mask_tok * (kt <= qt).astype(f32)

    s_full = jnp.dot(q.astype(bf), k.T.astype(bf),
                     preferred_element_type=f32) * scale
    p = s_full * jax.nn.sigmoid(s_full) * mask_tok
    out_slc = jnp.dot(p.astype(bf), v.astype(bf),
                      preferred_element_type=f32) * gs

    p_cmp = jnp.where(causal_blk, s_cmp * jax.nn.sigmoid(s_cmp), 0.0)
    out_cmp = jnp.dot(p_cmp.astype(bf), v_cmp.astype(bf),
                      preferred_element_type=f32) * gc

    out_ref[0, 0] = out_cmp + out_slc


def kernel(q, k, v, g_cmp, g_slc, x_offsets):
    T, H, D = q.shape
    Bn = x_offsets.shape[0] - 1
    L = T // Bn

    q4 = q.reshape(Bn, L, H, D).transpose(0, 2, 1, 3)
    k4 = k.reshape(Bn, L, H, D).transpose(0, 2, 1, 3)
    v4 = v.reshape(Bn, L, H, D).transpose(0, 2, 1, 3)
    gc = g_cmp.reshape(Bn, L, H, 1).transpose(0, 2, 1, 3)
    gs = g_slc.reshape(Bn, L, H, 1).transpose(0, 2, 1, 3)

    qkv_spec = pl.BlockSpec((1, 1, L, D), lambda b, h: (b, h, 0, 0))
    g_spec = pl.BlockSpec((1, 1, L, 1), lambda b, h: (b, h, 0, 0))

    body = functools.partial(_hstu_bsa_kernel, bs=_BLOCK_SIZE,
                             ssel=min(_BLOCK_COUNTS, L // _BLOCK_SIZE))
    out = pl.pallas_call(
        body,
        grid=(Bn, H),
        in_specs=[qkv_spec, qkv_spec, qkv_spec, g_spec, g_spec],
        out_specs=qkv_spec,
        out_shape=jax.ShapeDtypeStruct((Bn, H, L, D), jnp.float32),
        compiler_params=pltpu.CompilerParams(
            dimension_semantics=("parallel", "parallel")),
    )(q4, k4, v4, gc, gs)

    return out.transpose(0, 2, 1, 3).reshape(T, H, D)
